# baseline (speedup 1.0000x reference)
import functools

import jax
import jax.numpy as jnp
from jax import lax
from jax.experimental import pallas as pl
from jax.experimental.pallas import tpu as pltpu
from jax.experimental.pallas import tpu_sc as plsc

_NUM_EXPERTS = 8
_TOP_K = 2
_BLOCK_S = 1024
_LANES = 16


def _router_body(x_ref, w_ref, b_ref, logits_ref, aff_ref):
    bs, bdim, h = x_ref.shape
    x = x_ref[...].reshape(bs * bdim, h)
    w = w_ref[...]
    logits = jnp.dot(x, w, preferred_element_type=jnp.float32) + b_ref[...]
    lt = logits.T
    logits_ref[...] = lt

    m = jnp.max(lt, axis=0, keepdims=True)
    e = jnp.exp(lt - m)
    aff_ref[...] = e / jnp.sum(e, axis=0, keepdims=True)


def _tc_router(hidden_states, wt, b2):
    S, B, H = hidden_states.shape
    T = S * B
    block_t = _BLOCK_S * B
    grid = (S // _BLOCK_S,)
    return pl.pallas_call(
        _router_body,
        grid=grid,
        in_specs=[
            pl.BlockSpec((_BLOCK_S, B, H), lambda i: (i, 0, 0)),
            pl.BlockSpec((H, _NUM_EXPERTS), lambda i: (0, 0)),
            pl.BlockSpec((1, _NUM_EXPERTS), lambda i: (0, 0)),
        ],
        out_specs=[
            pl.BlockSpec((_NUM_EXPERTS, block_t), lambda i: (0, i)),
            pl.BlockSpec((_NUM_EXPERTS, block_t), lambda i: (0, i)),
        ],
        out_shape=[
            jax.ShapeDtypeStruct((_NUM_EXPERTS, T), jnp.float32),
            jax.ShapeDtypeStruct((_NUM_EXPERTS, T), jnp.float32),
        ],
    )(hidden_states, wt, b2)


def _make_sc_topk(T):
    info = plsc.get_sparse_core_info()
    nworkers = info.num_cores * info.num_subcores
    cols = T // nworkers
    mesh = plsc.VectorSubcoreMesh(core_axis_name="c", subcore_axis_name="s")

    @functools.partial(
        pl.kernel,
        mesh=mesh,
        out_type=jax.ShapeDtypeStruct((_TOP_K, T), jnp.int32),
        scratch_types=[
            pltpu.VMEM((_NUM_EXPERTS, cols), jnp.float32),
            pltpu.VMEM((_TOP_K, cols), jnp.int32),
        ],
    )
    def sc_topk(logits_hbm, out_hbm, lbuf, obuf):
        wid = lax.axis_index("s") * info.num_cores + lax.axis_index("c")
        base = wid * cols
        for e in range(_NUM_EXPERTS):
            pltpu.sync_copy(logits_hbm.at[e, pl.ds(base, cols)], lbuf.at[e])

        def body(j, carry):
            sl = pl.ds(j * _LANES, _LANES)
            v = [lbuf[e, sl] for e in range(_NUM_EXPERTS)]
            neg_inf = jnp.full((_LANES,), -jnp.inf, jnp.float32)
            best = v[0]
            bi = jnp.zeros((_LANES,), jnp.int32)
            for e in range(1, _NUM_EXPERTS):
                gt = v[e] > best
                bi = jnp.where(gt, jnp.int32(e), bi)
                best = jnp.maximum(v[e], best)
            second = neg_inf
            si = jnp.zeros((_LANES,), jnp.int32)
            for e in range(_NUM_EXPERTS):
                ve = jnp.where(bi == e, neg_inf, v[e])
                gt = ve > second
                si = jnp.where(gt, jnp.int32(e), si)
                second = jnp.maximum(ve, second)
            obuf[0, sl] = bi
            obuf[1, sl] = si
            return carry

        lax.fori_loop(0, cols // _LANES, body, 0)
        for r in range(_TOP_K):
            pltpu.sync_copy(obuf.at[r], out_hbm.at[r, pl.ds(base, cols)])

    return sc_topk


def kernel(hidden_states, W, b):
    S, B, H = hidden_states.shape
    T = S * B
    wt = W.T
    b2 = b.reshape(1, _NUM_EXPERTS)
    logits_t, aff_t = _tc_router(hidden_states, wt, b2)
    idx_t = _make_sc_topk(T)(logits_t)
    return (logits_t.T, aff_t.T, idx_t.T)


# speedup vs baseline: 1.6211x; 1.6211x over previous
import jax
import jax.numpy as jnp
from jax.experimental import pallas as pl

_NUM_EXPERTS = 8
_TOP_K = 2
_BLOCK_S = 1024


def _router_body(x_ref, w_ref, b_ref, logits_ref, aff_ref, idx_ref):
    bs, bdim, h = x_ref.shape
    x = x_ref[...].reshape(bs * bdim, h)
    w = w_ref[...]
    logits = jnp.dot(x, w, preferred_element_type=jnp.float32) + b_ref[...]
    lt = logits.T
    logits_ref[...] = lt

    m = jnp.max(lt, axis=0, keepdims=True)
    e = jnp.exp(lt - m)
    aff_ref[...] = e / jnp.sum(e, axis=0, keepdims=True)

    iota = jax.lax.broadcasted_iota(jnp.int32, lt.shape, 0)
    sentinel = jnp.int32(_NUM_EXPERTS)
    idx1 = jnp.min(jnp.where(lt == m, iota, sentinel), axis=0, keepdims=True)
    masked = jnp.where(iota == idx1, -jnp.inf, lt)
    m2 = jnp.max(masked, axis=0, keepdims=True)
    idx2 = jnp.min(jnp.where(masked == m2, iota, sentinel), axis=0, keepdims=True)
    idx_ref[...] = jnp.concatenate([idx1, idx2], axis=0)


def kernel(hidden_states, W, b):
    S, B, H = hidden_states.shape
    T = S * B
    block_t = _BLOCK_S * B
    wt = W.T
    b2 = b.reshape(1, _NUM_EXPERTS)

    grid = (S // _BLOCK_S,)
    logits_t, aff_t, idx_t = pl.pallas_call(
        _router_body,
        grid=grid,
        in_specs=[
            pl.BlockSpec((_BLOCK_S, B, H), lambda i: (i, 0, 0)),
            pl.BlockSpec((H, _NUM_EXPERTS), lambda i: (0, 0)),
            pl.BlockSpec((1, _NUM_EXPERTS), lambda i: (0, 0)),
        ],
        out_specs=[
            pl.BlockSpec((_NUM_EXPERTS, block_t), lambda i: (0, i)),
            pl.BlockSpec((_NUM_EXPERTS, block_t), lambda i: (0, i)),
            pl.BlockSpec((_TOP_K, block_t), lambda i: (0, i)),
        ],
        out_shape=[
            jax.ShapeDtypeStruct((_NUM_EXPERTS, T), jnp.float32),
            jax.ShapeDtypeStruct((_NUM_EXPERTS, T), jnp.float32),
            jax.ShapeDtypeStruct((_TOP_K, T), jnp.int32),
        ],
    )(hidden_states, wt, b2)
    return (logits_t.T, aff_t.T, idx_t.T)
